# Initial kernel scaffold; baseline (speedup 1.0000x reference)
import jax
import jax.numpy as jnp
from jax.experimental import pallas as pl


def kernel(x, table):
    raise NotImplementedError("write your pallas kernel here")


# speedup vs baseline: 3.3863x; 3.3863x over previous
import jax
import jax.numpy as jnp
from jax.experimental import pallas as pl

L_BLK = 1024


def _add_kernel(x_ref, t_ref, o_ref):
    o_ref[...] = x_ref[...] + t_ref[...][None, :, :]


def kernel(x, table):
    b, l, d = x.shape
    grid = (l // L_BLK, b)
    return pl.pallas_call(
        _add_kernel,
        grid=grid,
        in_specs=[
            pl.BlockSpec((1, L_BLK, d), lambda i, j: (j, i, 0)),
            pl.BlockSpec((L_BLK, d), lambda i, j: (i, 0)),
        ],
        out_specs=pl.BlockSpec((1, L_BLK, d), lambda i, j: (j, i, 0)),
        out_shape=jax.ShapeDtypeStruct((b, l, d), x.dtype),
    )(x, table)


# speedup vs baseline: 3.6184x; 1.0685x over previous
import jax
import jax.numpy as jnp
from jax.experimental import pallas as pl

L_BLK = 2048


def _add_kernel(x_ref, t_ref, o_ref):
    o_ref[...] = x_ref[...] + t_ref[...][None, :, :]


def kernel(x, table):
    b, l, d = x.shape
    grid = (l // L_BLK, b)
    return pl.pallas_call(
        _add_kernel,
        grid=grid,
        in_specs=[
            pl.BlockSpec((1, L_BLK, d), lambda i, j: (j, i, 0)),
            pl.BlockSpec((L_BLK, d), lambda i, j: (i, 0)),
        ],
        out_specs=pl.BlockSpec((1, L_BLK, d), lambda i, j: (j, i, 0)),
        out_shape=jax.ShapeDtypeStruct((b, l, d), x.dtype),
    )(x, table)
